# baseline (speedup 1.0000x reference)
import jax
import jax.numpy as jnp
from jax.experimental import pallas as pl
from jax.experimental.pallas import tpu as pltpu

_B = 16
_N = 100000
_R = 200
_L = 500
_L4 = 4 * _L
_RB = 8
_STEPS = _R // _RB
_SCALE_XY = 10.0
_SCALE_WH = 5.0
_NEG_RATIO = 3.0
_EPS = 1.1920928955078125e-07


def _smooth_l1(d):
    ad = jnp.abs(d)
    return jnp.where(ad < 1.0, 0.5 * d * d, ad - 0.5)


def _rowsum(x):
    return jnp.sum(jnp.sum(x, axis=2), axis=1)[:, None]


def _loss_kernel(p4, g4, plab, glab, u4, v4, out_ref,
                 bits_ref, ll_ref, e_ref,
                 pos_ref, bb_ref, tot_ref, pbce_ref, sr_ref):
    i = pl.program_id(0)

    @pl.when(i == 0)
    def _init():
        pos_ref[...] = jnp.zeros_like(pos_ref)
        bb_ref[...] = jnp.zeros_like(bb_ref)
        tot_ref[...] = jnp.zeros_like(tot_ref)
        pbce_ref[...] = jnp.zeros_like(pbce_ref)
        sr_ref[...] = jnp.zeros_like(sr_ref)
        li = jax.lax.broadcasted_iota(jnp.int32, (_L, _L4), 1) // 4
        ei = jax.lax.broadcasted_iota(jnp.int32, (_L, _L4), 0)
        e_ref[...] = (li == ei).astype(jnp.bfloat16)

    m = (glab[...] > 0.0).astype(jnp.float32)

    lane4 = jax.lax.broadcasted_iota(jnp.int32, (_B, _RB, _L4), 2)
    isxy = (lane4 % 4) < 2
    z = g4[...] * u4[...] + v4[...]
    t = jnp.where(isxy, z, _SCALE_WH * jnp.log(z))
    sl1 = _smooth_l1(p4[...] - t)

    m4 = jax.lax.dot_general(
        m.reshape(_B * _RB, _L).astype(jnp.bfloat16), e_ref[...],
        (((1,), (0,)), ((), ())),
        preferred_element_type=jnp.float32).reshape(_B, _RB, _L4)
    bb_part = _rowsum(m4 * sl1)

    x = plab[...]
    bce = (jnp.maximum(x, 0.0) - x * glab[...]
           + jnp.log(1.0 + jnp.exp(-jnp.abs(x))))
    lneg = jnp.where(m > 0.0, 0.0, bce)

    sl = pl.ds(i * _RB, _RB)
    bits_ref[:, sl, :] = jax.lax.bitcast_convert_type(lneg, jnp.int32)
    ll_ref[:, sl, :] = bce

    pos_ref[...] = pos_ref[...] + _rowsum(m)
    bb_ref[...] = bb_ref[...] + bb_part
    tot_ref[...] = tot_ref[...] + _rowsum(bce)
    pbce_ref[...] = pbce_ref[...] + _rowsum(m * bce)

    @pl.when(i == _STEPS - 1)
    def _finish():
        pos = pos_ref[...]
        k = jnp.minimum(_NEG_RATIO * pos, float(_N))
        neg = jnp.where(k >= float(_N), tot_ref[...], sr_ref[...])

        num_mask = (pos > 0.0).astype(jnp.float32)
        pos_f = jnp.maximum(pos, _EPS)
        w = num_mask / pos_f
        lb_s = jnp.sum(bb_ref[...] * w) / _B
        ll_s = jnp.sum((pbce_ref[...] + neg) * w) / _B
        total = (lb_s + ll_s) * (jnp.sum(w) / _B)
        lane = jax.lax.broadcasted_iota(jnp.int32, (1, 128), 1)
        vals = jnp.where(lane == 0, total,
                         jnp.where(lane == 1, lb_s,
                                   jnp.where(lane == 2, ll_s, 0.0)))
        out_ref[...] = vals


@jax.jit
def kernel(pbboxs, plabels, gbboxs, glabels, ancs):
    a_xy = ancs[:, 0:2]
    a_wh = ancs[:, 2:4]
    u = jnp.concatenate([_SCALE_XY / a_wh, 1.0 / a_wh], axis=1)
    v = jnp.concatenate([-_SCALE_XY * a_xy / a_wh, jnp.zeros_like(a_wh)],
                        axis=1)
    u4 = u.reshape(1, _R, _L4)
    v4 = v.reshape(1, _R, _L4)

    big_spec = pl.BlockSpec((_B, _RB, _L4), lambda i: (0, i, 0))
    lab_spec = pl.BlockSpec((_B, _RB, _L), lambda i: (0, i, 0))
    anc_spec = pl.BlockSpec((1, _RB, _L4), lambda i: (0, i, 0))

    out = pl.pallas_call(
        _loss_kernel,
        grid=(_STEPS,),
        in_specs=[big_spec, big_spec, lab_spec, lab_spec, anc_spec, anc_spec],
        out_specs=pl.BlockSpec((1, 128), lambda i: (0, 0)),
        out_shape=jax.ShapeDtypeStruct((1, 128), jnp.float32),
        scratch_shapes=[
            pltpu.VMEM((_B, _R, _L), jnp.int32),
            pltpu.VMEM((_B, _R, _L), jnp.float32),
            pltpu.VMEM((_L, _L4), jnp.bfloat16),
            pltpu.VMEM((_B, 1), jnp.float32),
            pltpu.VMEM((_B, 1), jnp.float32),
            pltpu.VMEM((_B, 1), jnp.float32),
            pltpu.VMEM((_B, 1), jnp.float32),
            pltpu.VMEM((_B, 1), jnp.float32),
        ],
    )(pbboxs.reshape(_B, _R, _L4), gbboxs.reshape(_B, _R, _L4),
      plabels.reshape(_B, _R, _L), glabels.reshape(_B, _R, _L), u4, v4)
    return (out[0, 0], out[0, 1], out[0, 2])


# speedup vs baseline: 3.4539x; 3.4539x over previous
import jax
import jax.numpy as jnp
from jax.experimental import pallas as pl
from jax.experimental.pallas import tpu as pltpu

_B = 16
_N = 100000
_R = 200
_L = 500
_RB = 8
_STEPS = _R // _RB
_SCALE_XY = 10.0
_SCALE_WH = 5.0
_NEG_RATIO = 3.0
_EPS = 1.1920928955078125e-07


def _smooth_l1(d):
    ad = jnp.abs(d)
    return jnp.where(ad < 1.0, 0.5 * d * d, ad - 0.5)


def _rowsum(x):
    return jnp.sum(jnp.sum(x, axis=2), axis=1)[:, None]


def _loss_kernel(p, g, plab, glab, a, out_ref,
                 bits_ref, ll_ref, pos_ref, bb_ref, tot_ref, pbce_ref,
                 sr_ref):
    i = pl.program_id(0)

    @pl.when(i == 0)
    def _init():
        pos_ref[...] = jnp.zeros_like(pos_ref)
        bb_ref[...] = jnp.zeros_like(bb_ref)
        tot_ref[...] = jnp.zeros_like(tot_ref)
        pbce_ref[...] = jnp.zeros_like(pbce_ref)
        sr_ref[...] = jnp.zeros_like(sr_ref)

    m = (glab[...] > 0.0).astype(jnp.float32)

    aw = a[0, 2]
    ah = a[0, 3]
    dx = p[:, 0] - _SCALE_XY * (g[:, 0] - a[0, 0]) / aw
    dy = p[:, 1] - _SCALE_XY * (g[:, 1] - a[0, 1]) / ah
    dw = p[:, 2] - _SCALE_WH * jnp.log(g[:, 2] / aw)
    dh = p[:, 3] - _SCALE_WH * jnp.log(g[:, 3] / ah)
    sl1 = _smooth_l1(dx) + _smooth_l1(dy) + _smooth_l1(dw) + _smooth_l1(dh)

    x = plab[...]
    bce = (jnp.maximum(x, 0.0) - x * glab[...]
           + jnp.log(1.0 + jnp.exp(-jnp.abs(x))))
    lneg = jnp.where(m > 0.0, 0.0, bce)

    sl = pl.ds(i * _RB, _RB)
    bits_ref[:, sl, :] = jax.lax.bitcast_convert_type(lneg, jnp.int32)
    ll_ref[:, sl, :] = bce

    pos_ref[...] = pos_ref[...] + _rowsum(m)
    bb_ref[...] = bb_ref[...] + _rowsum(m * sl1)
    tot_ref[...] = tot_ref[...] + _rowsum(bce)
    pbce_ref[...] = pbce_ref[...] + _rowsum(m * bce)

    @pl.when(i == _STEPS - 1)
    def _finish():
        pos = pos_ref[...]
        k = jnp.minimum(_NEG_RATIO * pos, float(_N))
        need = jnp.any((pos > 0.0) & (k < float(_N)))

        @pl.when(need)
        def _search():
            bits = bits_ref[...]
            ll = ll_ref[...]

            def cnt3(mask):
                return jnp.sum(jnp.sum(mask.astype(jnp.float32), axis=2),
                               axis=1)[:, None]

            def vstep(sh, lohi):
                lo, hi = lohi
                mid = lo + jax.lax.shift_right_logical(hi - lo + 1, 1)
                ok = cnt3(bits >= mid[:, :, None]) >= k
                return jnp.where(ok, mid, lo), jnp.where(ok, hi, mid - 1)

            lo0 = jnp.zeros((_B, 1), jnp.int32)
            hi0 = jnp.full((_B, 1), jnp.int32(0x7FFFFFFF))
            t, _ = jax.lax.fori_loop(0, 31, vstep, (lo0, hi0))

            t3 = t[:, :, None]
            gt = bits > t3
            sum_gt = jnp.sum(jnp.sum(jnp.where(gt, ll, 0.0), axis=2),
                             axis=1)[:, None]
            r = k - cnt3(gt)
            eq = bits == t3
            idx = (jax.lax.broadcasted_iota(jnp.int32, (_B, _R, _L), 1) * _L
                   + jax.lax.broadcasted_iota(jnp.int32, (_B, _R, _L), 2))

            def istep(sh, lohi):
                lo, hi = lohi
                mid = jax.lax.shift_right_logical(lo + hi, 1)
                ok = cnt3(eq & (idx < mid[:, :, None])) >= r
                return jnp.where(ok, lo, mid + 1), jnp.where(ok, mid, hi)

            ilo = jnp.zeros((_B, 1), jnp.int32)
            ihi = jnp.full((_B, 1), jnp.int32(_N))
            mth, _ = jax.lax.fori_loop(0, 18, istep, (ilo, ihi))

            tie = jnp.sum(jnp.sum(
                jnp.where(eq & (idx < mth[:, :, None]), ll, 0.0),
                axis=2), axis=1)[:, None]
            sr_ref[...] = sum_gt + tie

        neg = jnp.where(k >= float(_N), tot_ref[...], sr_ref[...])

        num_mask = (pos > 0.0).astype(jnp.float32)
        pos_f = jnp.maximum(pos, _EPS)
        w = num_mask / pos_f
        lb_s = jnp.sum(bb_ref[...] * w) / _B
        ll_s = jnp.sum((pbce_ref[...] + neg) * w) / _B
        total = (lb_s + ll_s) * (jnp.sum(w) / _B)
        lane = jax.lax.broadcasted_iota(jnp.int32, (1, 128), 1)
        vals = jnp.where(lane == 0, total,
                         jnp.where(lane == 1, lb_s,
                                   jnp.where(lane == 2, ll_s, 0.0)))
        out_ref[...] = vals


@jax.jit
def kernel(pbboxs, plabels, gbboxs, glabels, ancs):
    pT = jnp.transpose(pbboxs, (0, 2, 1)).reshape(_B, 4, _R, _L)
    gT = jnp.transpose(gbboxs, (0, 2, 1)).reshape(_B, 4, _R, _L)
    aT = jnp.transpose(ancs, (1, 0)).reshape(1, 4, _R, _L)

    box_spec = pl.BlockSpec((_B, 4, _RB, _L), lambda i: (0, 0, i, 0))
    lab_spec = pl.BlockSpec((_B, _RB, _L), lambda i: (0, i, 0))
    anc_spec = pl.BlockSpec((1, 4, _RB, _L), lambda i: (0, 0, i, 0))

    out = pl.pallas_call(
        _loss_kernel,
        grid=(_STEPS,),
        in_specs=[box_spec, box_spec, lab_spec, lab_spec, anc_spec],
        out_specs=pl.BlockSpec((1, 128), lambda i: (0, 0)),
        out_shape=jax.ShapeDtypeStruct((1, 128), jnp.float32),
        scratch_shapes=[
            pltpu.VMEM((_B, _R, _L), jnp.int32),
            pltpu.VMEM((_B, _R, _L), jnp.float32),
            pltpu.VMEM((_B, 1), jnp.float32),
            pltpu.VMEM((_B, 1), jnp.float32),
            pltpu.VMEM((_B, 1), jnp.float32),
            pltpu.VMEM((_B, 1), jnp.float32),
            pltpu.VMEM((_B, 1), jnp.float32),
        ],
    )(pT, gT, plabels.reshape(_B, _R, _L), glabels.reshape(_B, _R, _L), aT)
    return (out[0, 0], out[0, 1], out[0, 2])


# speedup vs baseline: 3.6923x; 1.0690x over previous
import jax
import jax.numpy as jnp
from jax.experimental import pallas as pl
from jax.experimental.pallas import tpu as pltpu

_B = 16
_N = 100000
_R = 200
_L = 500
_RB = 40
_STEPS = _R // _RB
_SCALE_XY = 10.0
_SCALE_WH = 5.0
_NEG_RATIO = 3.0
_EPS = 1.1920928955078125e-07


def _smooth_l1(d):
    ad = jnp.abs(d)
    return jnp.where(ad < 1.0, 0.5 * d * d, ad - 0.5)


def _rowsum(x):
    return jnp.sum(jnp.sum(x, axis=2), axis=1)[:, None]


def _loss_kernel(p, g, plab, glab, a, out_ref,
                 bits_ref, ll_ref, pos_ref, bb_ref, tot_ref, pbce_ref,
                 sr_ref):
    i = pl.program_id(0)

    @pl.when(i == 0)
    def _init():
        pos_ref[...] = jnp.zeros_like(pos_ref)
        bb_ref[...] = jnp.zeros_like(bb_ref)
        tot_ref[...] = jnp.zeros_like(tot_ref)
        pbce_ref[...] = jnp.zeros_like(pbce_ref)
        sr_ref[...] = jnp.zeros_like(sr_ref)

    m = (glab[...] > 0.0).astype(jnp.float32)

    aw = a[0, 2]
    ah = a[0, 3]
    dx = p[:, 0] - _SCALE_XY * (g[:, 0] - a[0, 0]) / aw
    dy = p[:, 1] - _SCALE_XY * (g[:, 1] - a[0, 1]) / ah
    dw = p[:, 2] - _SCALE_WH * jnp.log(g[:, 2] / aw)
    dh = p[:, 3] - _SCALE_WH * jnp.log(g[:, 3] / ah)
    sl1 = _smooth_l1(dx) + _smooth_l1(dy) + _smooth_l1(dw) + _smooth_l1(dh)

    x = plab[...]
    bce = (jnp.maximum(x, 0.0) - x * glab[...]
           + jnp.log(1.0 + jnp.exp(-jnp.abs(x))))
    lneg = jnp.where(m > 0.0, 0.0, bce)

    sl = pl.ds(i * _RB, _RB)
    bits_ref[:, sl, :] = jax.lax.bitcast_convert_type(lneg, jnp.int32)
    ll_ref[:, sl, :] = bce

    pos_ref[...] = pos_ref[...] + _rowsum(m)
    bb_ref[...] = bb_ref[...] + _rowsum(m * sl1)
    tot_ref[...] = tot_ref[...] + _rowsum(bce)
    pbce_ref[...] = pbce_ref[...] + _rowsum(m * bce)

    @pl.when(i == _STEPS - 1)
    def _finish():
        pos = pos_ref[...]
        k = jnp.minimum(_NEG_RATIO * pos, float(_N))
        need = jnp.any((pos > 0.0) & (k < float(_N)))

        @pl.when(need)
        def _search():
            bits = bits_ref[...]
            ll = ll_ref[...]

            def cnt3(mask):
                return jnp.sum(jnp.sum(mask.astype(jnp.float32), axis=2),
                               axis=1)[:, None]

            def vstep(sh, lohi):
                lo, hi = lohi
                mid = lo + jax.lax.shift_right_logical(hi - lo + 1, 1)
                ok = cnt3(bits >= mid[:, :, None]) >= k
                return jnp.where(ok, mid, lo), jnp.where(ok, hi, mid - 1)

            lo0 = jnp.zeros((_B, 1), jnp.int32)
            hi0 = jnp.full((_B, 1), jnp.int32(0x7FFFFFFF))
            t, _ = jax.lax.fori_loop(0, 31, vstep, (lo0, hi0))

            t3 = t[:, :, None]
            gt = bits > t3
            sum_gt = jnp.sum(jnp.sum(jnp.where(gt, ll, 0.0), axis=2),
                             axis=1)[:, None]
            r = k - cnt3(gt)
            eq = bits == t3
            idx = (jax.lax.broadcasted_iota(jnp.int32, (_B, _R, _L), 1) * _L
                   + jax.lax.broadcasted_iota(jnp.int32, (_B, _R, _L), 2))

            def istep(sh, lohi):
                lo, hi = lohi
                mid = jax.lax.shift_right_logical(lo + hi, 1)
                ok = cnt3(eq & (idx < mid[:, :, None])) >= r
                return jnp.where(ok, lo, mid + 1), jnp.where(ok, mid, hi)

            ilo = jnp.zeros((_B, 1), jnp.int32)
            ihi = jnp.full((_B, 1), jnp.int32(_N))
            mth, _ = jax.lax.fori_loop(0, 18, istep, (ilo, ihi))

            tie = jnp.sum(jnp.sum(
                jnp.where(eq & (idx < mth[:, :, None]), ll, 0.0),
                axis=2), axis=1)[:, None]
            sr_ref[...] = sum_gt + tie

        neg = jnp.where(k >= float(_N), tot_ref[...], sr_ref[...])

        num_mask = (pos > 0.0).astype(jnp.float32)
        pos_f = jnp.maximum(pos, _EPS)
        w = num_mask / pos_f
        lb_s = jnp.sum(bb_ref[...] * w) / _B
        ll_s = jnp.sum((pbce_ref[...] + neg) * w) / _B
        total = (lb_s + ll_s) * (jnp.sum(w) / _B)
        lane = jax.lax.broadcasted_iota(jnp.int32, (1, 128), 1)
        vals = jnp.where(lane == 0, total,
                         jnp.where(lane == 1, lb_s,
                                   jnp.where(lane == 2, ll_s, 0.0)))
        out_ref[...] = vals


@jax.jit
def kernel(pbboxs, plabels, gbboxs, glabels, ancs):
    pT = jnp.transpose(pbboxs, (0, 2, 1)).reshape(_B, 4, _R, _L)
    gT = jnp.transpose(gbboxs, (0, 2, 1)).reshape(_B, 4, _R, _L)
    aT = jnp.transpose(ancs, (1, 0)).reshape(1, 4, _R, _L)

    box_spec = pl.BlockSpec((_B, 4, _RB, _L), lambda i: (0, 0, i, 0))
    lab_spec = pl.BlockSpec((_B, _RB, _L), lambda i: (0, i, 0))
    anc_spec = pl.BlockSpec((1, 4, _RB, _L), lambda i: (0, 0, i, 0))

    out = pl.pallas_call(
        _loss_kernel,
        grid=(_STEPS,),
        in_specs=[box_spec, box_spec, lab_spec, lab_spec, anc_spec],
        out_specs=pl.BlockSpec((1, 128), lambda i: (0, 0)),
        out_shape=jax.ShapeDtypeStruct((1, 128), jnp.float32),
        scratch_shapes=[
            pltpu.VMEM((_B, _R, _L), jnp.int32),
            pltpu.VMEM((_B, _R, _L), jnp.float32),
            pltpu.VMEM((_B, 1), jnp.float32),
            pltpu.VMEM((_B, 1), jnp.float32),
            pltpu.VMEM((_B, 1), jnp.float32),
            pltpu.VMEM((_B, 1), jnp.float32),
            pltpu.VMEM((_B, 1), jnp.float32),
        ],
    )(pT, gT, plabels.reshape(_B, _R, _L), glabels.reshape(_B, _R, _L), aT)
    return (out[0, 0], out[0, 1], out[0, 2])


# speedup vs baseline: 3.9813x; 1.0783x over previous
import jax
import jax.numpy as jnp
from jax.experimental import pallas as pl
from jax.experimental.pallas import tpu as pltpu

_B = 16
_N = 100000
_GB = 2
_STEPS = _B // _GB
_SCALE_XY = 10.0
_SCALE_WH = 5.0
_NEG_RATIO = 3.0
_EPS = 1.1920928955078125e-07


def _smooth_l1(d):
    ad = jnp.abs(d)
    return jnp.where(ad < 1.0, 0.5 * d * d, ad - 0.5)


def _loss_kernel(p, g, plab, glab, u, v, out_ref,
                 bits_ref, ll_ref, pos_ref, bb_ref, pbce_ref, neg_ref):
    i = pl.program_id(0)
    rows = pl.ds(i * _GB, _GB)

    m = (glab[...][:, 0, :] > 0.0).astype(jnp.float32)

    comp = jax.lax.broadcasted_iota(jnp.int32, (_GB, 4, _N), 1)
    z = g[...] * u[...] + v[...]
    t = jnp.where(comp < 2, z, _SCALE_WH * jnp.log(z))
    sl1 = jnp.sum(_smooth_l1(p[...] - t), axis=1)

    x = plab[...][:, 0, :]
    bce = (jnp.maximum(x, 0.0) - x * glab[...][:, 0, :]
           + jnp.log(1.0 + jnp.exp(-jnp.abs(x))))
    lneg = jnp.where(m > 0.0, 0.0, bce)

    bits_ref[...] = jax.lax.bitcast_convert_type(lneg, jnp.int32)
    ll_ref[...] = bce

    pos = jnp.sum(m, axis=1, keepdims=True)
    tot = jnp.sum(bce, axis=1, keepdims=True)
    pos_ref[rows, :] = pos
    bb_ref[rows, :] = jnp.sum(m * sl1, axis=1, keepdims=True)
    pbce_ref[rows, :] = jnp.sum(m * bce, axis=1, keepdims=True)

    k = jnp.minimum(_NEG_RATIO * pos, float(_N))
    need = jnp.any((pos > 0.0) & (k < float(_N)))
    neg_ref[rows, :] = jnp.where(k >= float(_N), tot, 0.0)

    @pl.when(need)
    def _search():
        bits = bits_ref[...]
        ll = ll_ref[...]

        def cnt(mask):
            return jnp.sum(mask.astype(jnp.float32), axis=1, keepdims=True)

        def vstep(sh, lohi):
            lo, hi = lohi
            mid = lo + jax.lax.shift_right_logical(hi - lo + 1, 1)
            ok = cnt(bits >= mid) >= k
            return jnp.where(ok, mid, lo), jnp.where(ok, hi, mid - 1)

        lo0 = jnp.zeros((_GB, 1), jnp.int32)
        hi0 = jnp.full((_GB, 1), jnp.int32(0x7FFFFFFF))
        th, _ = jax.lax.fori_loop(0, 31, vstep, (lo0, hi0))

        gt = bits > th
        sum_gt = jnp.sum(jnp.where(gt, ll, 0.0), axis=1, keepdims=True)
        r = k - cnt(gt)
        eq = bits == th
        idx = jax.lax.broadcasted_iota(jnp.int32, (_GB, _N), 1)

        def istep(sh, lohi):
            lo, hi = lohi
            mid = jax.lax.shift_right_logical(lo + hi, 1)
            ok = cnt(eq & (idx < mid)) >= r
            return jnp.where(ok, lo, mid + 1), jnp.where(ok, mid, hi)

        ilo = jnp.zeros((_GB, 1), jnp.int32)
        ihi = jnp.full((_GB, 1), jnp.int32(_N))
        mth, _ = jax.lax.fori_loop(0, 18, istep, (ilo, ihi))

        tie = jnp.sum(jnp.where(eq & (idx < mth), ll, 0.0), axis=1,
                      keepdims=True)
        searched = sum_gt + tie
        neg_ref[rows, :] = jnp.where(k >= float(_N), tot, searched)

    @pl.when(i == _STEPS - 1)
    def _finish():
        posf = pos_ref[...]
        num_mask = (posf > 0.0).astype(jnp.float32)
        pos_f = jnp.maximum(posf, _EPS)
        w = num_mask / pos_f
        lb_s = jnp.sum(bb_ref[...] * w) / _B
        ll_s = jnp.sum((pbce_ref[...] + neg_ref[...]) * w) / _B
        total = (lb_s + ll_s) * (jnp.sum(w) / _B)
        lane = jax.lax.broadcasted_iota(jnp.int32, (1, 128), 1)
        vals = jnp.where(lane == 0, total,
                         jnp.where(lane == 1, lb_s,
                                   jnp.where(lane == 2, ll_s, 0.0)))
        out_ref[...] = vals


@jax.jit
def kernel(pbboxs, plabels, gbboxs, glabels, ancs):
    pT = jnp.transpose(pbboxs, (0, 2, 1))
    gT = jnp.transpose(gbboxs, (0, 2, 1))
    aT = jnp.transpose(ancs, (1, 0))
    a_xy, a_wh = aT[0:2], aT[2:4]
    u = jnp.concatenate([_SCALE_XY / a_wh, 1.0 / a_wh], axis=0)[None]
    v = jnp.concatenate([-_SCALE_XY * a_xy / a_wh,
                         jnp.zeros_like(a_wh)], axis=0)[None]

    box_spec = pl.BlockSpec((_GB, 4, _N), lambda i: (i, 0, 0))
    lab_spec = pl.BlockSpec((_GB, 1, _N), lambda i: (i, 0, 0))
    anc_spec = pl.BlockSpec((1, 4, _N), lambda i: (0, 0, 0))

    out = pl.pallas_call(
        _loss_kernel,
        grid=(_STEPS,),
        in_specs=[box_spec, box_spec, lab_spec, lab_spec, anc_spec, anc_spec],
        out_specs=pl.BlockSpec((1, 128), lambda i: (0, 0)),
        out_shape=jax.ShapeDtypeStruct((1, 128), jnp.float32),
        scratch_shapes=[
            pltpu.VMEM((_GB, _N), jnp.int32),
            pltpu.VMEM((_GB, _N), jnp.float32),
            pltpu.VMEM((_B, 1), jnp.float32),
            pltpu.VMEM((_B, 1), jnp.float32),
            pltpu.VMEM((_B, 1), jnp.float32),
            pltpu.VMEM((_B, 1), jnp.float32),
        ],
    )(pT, gT, plabels[:, None, :], glabels[:, None, :], u, v)
    return (out[0, 0], out[0, 1], out[0, 2])


# speedup vs baseline: 4.2879x; 1.0770x over previous
import jax
import jax.numpy as jnp
from jax.experimental import pallas as pl
from jax.experimental.pallas import tpu as pltpu

_B = 16
_N = 100000
_GB = 4
_STEPS = _B // _GB
_SCALE_XY = 10.0
_SCALE_WH = 5.0
_NEG_RATIO = 3.0
_EPS = 1.1920928955078125e-07


def _smooth_l1(d):
    ad = jnp.abs(d)
    return jnp.where(ad < 1.0, 0.5 * d * d, ad - 0.5)


def _loss_kernel(p, g, plab, glab, u, v, out_ref,
                 bits_ref, ll_ref, pos_ref, bb_ref, pbce_ref, neg_ref):
    i = pl.program_id(0)
    rows = pl.ds(i * _GB, _GB)

    m = (glab[...][:, 0, :] > 0.0).astype(jnp.float32)

    sl1 = jnp.zeros((_GB, _N), jnp.float32)
    for j in range(4):
        zj = g[:, j] * u[:, j] + v[:, j]
        tj = zj if j < 2 else _SCALE_WH * jnp.log(zj)
        sl1 = sl1 + _smooth_l1(p[:, j] - tj)

    x = plab[...][:, 0, :]
    bce = (jnp.maximum(x, 0.0) - x * glab[...][:, 0, :]
           + jnp.log(1.0 + jnp.exp(-jnp.abs(x))))
    lneg = jnp.where(m > 0.0, 0.0, bce)

    bits_ref[...] = jax.lax.bitcast_convert_type(lneg, jnp.int32)
    ll_ref[...] = bce

    pos = jnp.sum(m, axis=1, keepdims=True)
    tot = jnp.sum(bce, axis=1, keepdims=True)
    pos_ref[rows, :] = pos
    bb_ref[rows, :] = jnp.sum(m * sl1, axis=1, keepdims=True)
    pbce_ref[rows, :] = jnp.sum(m * bce, axis=1, keepdims=True)

    k = jnp.minimum(_NEG_RATIO * pos, float(_N))
    need = jnp.any((pos > 0.0) & (k < float(_N)))
    neg_ref[rows, :] = jnp.where(k >= float(_N), tot, 0.0)

    @pl.when(need)
    def _search():
        bits = bits_ref[...]
        ll = ll_ref[...]

        def cnt(mask):
            return jnp.sum(mask.astype(jnp.float32), axis=1, keepdims=True)

        def vstep(sh, lohi):
            lo, hi = lohi
            mid = lo + jax.lax.shift_right_logical(hi - lo + 1, 1)
            ok = cnt(bits >= mid) >= k
            return jnp.where(ok, mid, lo), jnp.where(ok, hi, mid - 1)

        lo0 = jnp.zeros((_GB, 1), jnp.int32)
        hi0 = jnp.full((_GB, 1), jnp.int32(0x7FFFFFFF))
        th, _ = jax.lax.fori_loop(0, 31, vstep, (lo0, hi0))

        gt = bits > th
        sum_gt = jnp.sum(jnp.where(gt, ll, 0.0), axis=1, keepdims=True)
        r = k - cnt(gt)
        eq = bits == th
        idx = jax.lax.broadcasted_iota(jnp.int32, (_GB, _N), 1)

        def istep(sh, lohi):
            lo, hi = lohi
            mid = jax.lax.shift_right_logical(lo + hi, 1)
            ok = cnt(eq & (idx < mid)) >= r
            return jnp.where(ok, lo, mid + 1), jnp.where(ok, mid, hi)

        ilo = jnp.zeros((_GB, 1), jnp.int32)
        ihi = jnp.full((_GB, 1), jnp.int32(_N))
        mth, _ = jax.lax.fori_loop(0, 18, istep, (ilo, ihi))

        tie = jnp.sum(jnp.where(eq & (idx < mth), ll, 0.0), axis=1,
                      keepdims=True)
        searched = sum_gt + tie
        neg_ref[rows, :] = jnp.where(k >= float(_N), tot, searched)

    @pl.when(i == _STEPS - 1)
    def _finish():
        posf = pos_ref[...]
        num_mask = (posf > 0.0).astype(jnp.float32)
        pos_f = jnp.maximum(posf, _EPS)
        w = num_mask / pos_f
        lb_s = jnp.sum(bb_ref[...] * w) / _B
        ll_s = jnp.sum((pbce_ref[...] + neg_ref[...]) * w) / _B
        total = (lb_s + ll_s) * (jnp.sum(w) / _B)
        lane = jax.lax.broadcasted_iota(jnp.int32, (1, 128), 1)
        vals = jnp.where(lane == 0, total,
                         jnp.where(lane == 1, lb_s,
                                   jnp.where(lane == 2, ll_s, 0.0)))
        out_ref[...] = vals


@jax.jit
def kernel(pbboxs, plabels, gbboxs, glabels, ancs):
    pT = jnp.transpose(pbboxs, (0, 2, 1))
    gT = jnp.transpose(gbboxs, (0, 2, 1))
    aT = jnp.transpose(ancs, (1, 0))
    a_xy, a_wh = aT[0:2], aT[2:4]
    u = jnp.concatenate([_SCALE_XY / a_wh, 1.0 / a_wh], axis=0)[None]
    v = jnp.concatenate([-_SCALE_XY * a_xy / a_wh,
                         jnp.zeros_like(a_wh)], axis=0)[None]

    box_spec = pl.BlockSpec((_GB, 4, _N), lambda i: (i, 0, 0))
    lab_spec = pl.BlockSpec((_GB, 1, _N), lambda i: (i, 0, 0))
    anc_spec = pl.BlockSpec((1, 4, _N), lambda i: (0, 0, 0))

    out = pl.pallas_call(
        _loss_kernel,
        grid=(_STEPS,),
        in_specs=[box_spec, box_spec, lab_spec, lab_spec, anc_spec, anc_spec],
        out_specs=pl.BlockSpec((1, 128), lambda i: (0, 0)),
        out_shape=jax.ShapeDtypeStruct((1, 128), jnp.float32),
        scratch_shapes=[
            pltpu.VMEM((_GB, _N), jnp.int32),
            pltpu.VMEM((_GB, _N), jnp.float32),
            pltpu.VMEM((_B, 1), jnp.float32),
            pltpu.VMEM((_B, 1), jnp.float32),
            pltpu.VMEM((_B, 1), jnp.float32),
            pltpu.VMEM((_B, 1), jnp.float32),
        ],
    )(pT, gT, plabels[:, None, :], glabels[:, None, :], u, v)
    return (out[0, 0], out[0, 1], out[0, 2])
